# baseline (speedup 1.0000x reference)
import functools

import jax
import jax.numpy as jnp
from jax import lax
from jax.experimental import pallas as pl
from jax.experimental.pallas import tpu as pltpu


def _dysample_kernel(x_ref, w_ref, b_ref, o_ref, *, H, W, s, G):
    N = x_ref.shape[2]
    Cout = w_ref.shape[0]
    C = x_ref.shape[1]
    Cg = C // G
    sWs = s * W * s

    off = (jnp.dot(w_ref[...], x_ref[0], preferred_element_type=jnp.float32)
           + b_ref[...])
    nio = lax.broadcasted_iota(jnp.int32, (1, N), 1)
    ch = lax.broadcasted_iota(jnp.int32, (Cout, 1), 0)
    is_x = ((ch // G) % 2) == 0
    base = jnp.where(is_x, nio % W, nio // W).astype(jnp.float32)
    bound = jnp.where(is_x, float(W - 1), float(H - 1))
    pos = jnp.clip(off + base, 0.0, bound)

    lio = lax.broadcasted_iota(jnp.int32, (2 * G, sWs), 1)
    idx = ((lio // (s * W)) * (s * W) + (lio % s) * W
           + (lio % (s * W)) // s)
    frags = []
    for h in range(H):
        parts = [pos[k * 2 * G:(k + 1) * 2 * G, h * W:(h + 1) * W]
                 for k in range(s * s)]
        cat = jnp.concatenate(parts, axis=1)
        frags.append(jnp.take_along_axis(cat, idx, axis=1))
    pxy = jnp.concatenate(frags, axis=1)

    xb = x_ref[0].astype(jnp.bfloat16)
    S = H * sWs
    nchunk = 2 if S % 2 == 0 else 1
    tS = S // nchunk
    wio = lax.broadcasted_iota(jnp.int32, (W, tS), 0)
    hio = lax.broadcasted_iota(jnp.int32, (H, tS), 0)
    for g in range(G):
        for c in range(nchunk):
            sl = slice(c * tS, (c + 1) * tS)
            px = pxy[g:g + 1, sl]
            py = pxy[G + g:G + g + 1, sl]
            x0f = jnp.floor(px)
            y0f = jnp.floor(py)
            wx = px - x0f
            wy = py - y0f
            x0 = x0f.astype(jnp.int32)
            y0 = y0f.astype(jnp.int32)
            x1 = jnp.minimum(x0 + 1, W - 1)
            y1 = jnp.minimum(y0 + 1, H - 1)
            colT = (jnp.where(wio == x0, 1.0 - wx, 0.0)
                    + jnp.where(wio == x1, wx, 0.0)).astype(jnp.bfloat16)
            rowT = (jnp.where(hio == y0, 1.0 - wy, 0.0)
                    + jnp.where(hio == y1, wy, 0.0)).astype(jnp.bfloat16)
            m = (rowT[:, None, :] * colT[None, :, :]).reshape(H * W, tS)
            o_ref[0, g * Cg:(g + 1) * Cg, sl] = jnp.dot(
                xb[g * Cg:(g + 1) * Cg, :], m,
                preferred_element_type=jnp.float32)


def _dysample(x2, wp, bp, *, H, W, s, G):
    B, C, N = x2.shape
    Cout = wp.shape[0]
    S = s * s * N
    kern = functools.partial(_dysample_kernel, H=H, W=W, s=s, G=G)
    return pl.pallas_call(
        kern,
        out_shape=jax.ShapeDtypeStruct((B, C, S), jnp.float32),
        grid=(B,),
        in_specs=[
            pl.BlockSpec((1, C, N), lambda b: (b, 0, 0)),
            pl.BlockSpec((Cout, C), lambda b: (0, 0)),
            pl.BlockSpec((Cout, 1), lambda b: (0, 0)),
        ],
        out_specs=pl.BlockSpec((1, C, S), lambda b: (b, 0, 0)),
        compiler_params=pltpu.CompilerParams(
            dimension_semantics=("parallel",)),
    )(x2, wp, bp)


def _init_pos(scale, groups):
    h = (jnp.arange(scale, dtype=jnp.float32) - (scale - 1) / 2.0) / scale
    t0 = jnp.broadcast_to(h[None, :], (scale, scale))
    t1 = jnp.broadcast_to(h[:, None], (scale, scale))
    return jnp.tile(jnp.stack([t0, t1]), (1, groups, 1)).reshape(-1)


def kernel(x, weight, bias):
    B, C, H, W = x.shape
    Cout = weight.shape[0]
    G, s = 4, 2
    N = H * W

    perm = jnp.array([((ax * G + g) * s + sy) * s + sx
                      for sy in range(s) for sx in range(s)
                      for ax in range(2) for g in range(G)], dtype=jnp.int32)
    wp = (weight * 0.25).astype(jnp.float32)[perm]
    bp = (bias * 0.25 + _init_pos(s, G)).astype(jnp.float32)[perm]
    bp = bp.reshape(Cout, 1)

    x2 = x.reshape(B, C, N)
    samp = _dysample(x2, wp, bp, H=H, W=W, s=s, G=G)
    return samp.reshape(B, C, s * H, s * W)
